# baseline (speedup 1.0000x reference)
import functools

import jax
import jax.numpy as jnp
from jax import lax
from jax.experimental import pallas as pl
from jax.experimental.pallas import tpu as pltpu
from jax.experimental.pallas import tpu_sc as plsc

N = 50000
F = 5
ND = 2
H = 128
DISS = 0.01
DT = 1.0

NCORE = 2
NSUB = 16
NW = NCORE * NSUB
L = 16

K = 256
GPC = K // L
NBUF = 4
NACC = 50048
RPT = NACC // NSUB
XROW = 8


def _edge_sc_kernel(nchunk, ept, E, xpad, eidx, dummy, zrows, out, *bufs):
    IR = 2 * NBUF
    ibuf = bufs[0:IR]
    srows = bufs[IR:IR + NBUF]
    drows = bufs[IR + NBUF:IR + 2 * NBUF]
    semg = bufs[IR + 2 * NBUF:IR + 3 * NBUF]
    sems = bufs[IR + 3 * NBUF:IR + 4 * NBUF]
    semi = bufs[IR + 4 * NBUF:IR + 4 * NBUF + IR]
    acc = bufs[IR + 4 * NBUF + IR]

    c = lax.axis_index("c")
    s = lax.axis_index("s")
    wid = s * NCORE + c

    def issue_idx(ci, islot):
        base = wid * ept + ci * K
        @pl.when(base + K <= E)
        def _():
            pltpu.async_copy(eidx.at[:, pl.ds(base, K)], ibuf[islot],
                             semi[islot])
        @pl.when(base + K > E)
        def _():
            pltpu.async_copy(dummy, ibuf[islot], semi[islot])

    def wait_idx(islot):
        pltpu.make_async_copy(dummy, ibuf[islot], semi[islot]).wait()

    def issue_gathers(b, islot):
        pltpu.async_copy(xpad.at[ibuf[islot].at[0]], srows[b], semg[b])
        pltpu.async_copy(xpad.at[ibuf[islot].at[1]], drows[b], semg[b])

    def wait_gathers(b):
        pltpu.make_async_copy(xpad.at[ibuf[0].at[0]], srows[b], semg[b]).wait()
        pltpu.make_async_copy(xpad.at[ibuf[0].at[1]], drows[b], semg[b]).wait()

    def wait_scatter(b):
        pltpu.make_async_copy(srows[b], acc.at[ibuf[0].at[1]], sems[b]).wait()

    for j in range(NBUF):
        issue_idx(j, j)
    for j in range(2):
        wait_idx(j)
        issue_gathers(j, j)

    pltpu.sync_copy(zrows, acc.at[pl.ds(s * RPT, RPT)])
    plsc.subcore_barrier()

    def group_body(g, carry):
        for b in range(2 * NBUF):
            i = g * 2 * NBUF + b
            sb = b % NBUF
            wait_gathers(sb)
            for gg in range(GPC):
                rows = lax.iota(jnp.int32, L) + (gg * L)

                def col(ref, cc):
                    return plsc.load_gather(
                        ref, [rows, jnp.full((L,), cc, jnp.int32)])

                s0 = col(srows[sb], 0)
                s1 = col(srows[sb], 1)
                s3 = col(srows[sb], 3)
                s4 = col(srows[sb], 4)
                d0 = col(drows[sb], 0)
                d1 = col(drows[sb], 1)
                d3 = col(drows[sb], 3)
                d4 = col(drows[sb], 4)
                ex = s0 - d0
                ey = s1 - d1
                t = ex * ex + ey * ey
                bi = plsc.bitcast(t, jnp.int32)
                bi = jnp.int32(0x5F3759DF) - lax.shift_right_logical(bi, 1)
                y = plsc.bitcast(bi, jnp.float32)
                ht = t * 0.5
                y = y * (1.5 - ht * y * y)
                y = y * (1.5 - ht * y * y)
                y = y * (1.5 - ht * y * y)
                dist = t * y
                w = 1.0 / (dist + 1e-6)
                n0 = w * (s3 - d3)
                n1 = w * (s4 - d4)
                for cc, val in ((5, n0), (6, n1), (7, w)):
                    plsc.store_scatter(
                        srows[sb], [rows, jnp.full((L,), cc, jnp.int32)], val)
            pltpu.async_copy(srows[sb], acc.at[ibuf[b].at[1]], sems[sb],
                             add=True)
            if b >= 2:
                wait_scatter((b - 2) % NBUF)
            else:
                @pl.when(g > 0)
                def _():
                    wait_scatter((b - 2) % NBUF)
            @pl.when(i + 4 < nchunk)
            def _():
                issue_idx(i + 4, (b + 4) % IR)
            @pl.when(i + 2 < nchunk)
            def _():
                wait_idx((b + 2) % IR)
                issue_gathers((b + 2) % NBUF, (b + 2) % IR)
        return carry

    lax.fori_loop(0, nchunk // (2 * NBUF), group_body, 0)
    wait_scatter((nchunk - 2) % NBUF)
    wait_scatter((nchunk - 1) % NBUF)
    plsc.subcore_barrier()
    pltpu.sync_copy(acc.at[pl.ds(s * RPT, RPT)],
                    out.at[c].at[pl.ds(s * RPT, RPT)])


def _edge_pass(xpad, eidx, dummy, zrows, nchunk, ept, E):
    mesh = plsc.VectorSubcoreMesh(
        core_axis_name="c", subcore_axis_name="s",
        num_cores=NCORE, num_subcores=NSUB)
    IR = 2 * NBUF
    scratch = (
        [pltpu.VMEM((2, K), jnp.int32) for _ in range(IR)]
        + [pltpu.VMEM((K, XROW), jnp.float32) for _ in range(NBUF)]
        + [pltpu.VMEM((K, XROW), jnp.float32) for _ in range(NBUF)]
        + [pltpu.SemaphoreType.DMA for _ in range(NBUF)]
        + [pltpu.SemaphoreType.DMA for _ in range(NBUF)]
        + [pltpu.SemaphoreType.DMA for _ in range(IR)]
        + [pltpu.VMEM_SHARED((NACC, XROW), jnp.float32)]
    )
    kern = pl.kernel(
        functools.partial(_edge_sc_kernel, nchunk, ept, E),
        out_type=jax.ShapeDtypeStruct((NCORE, NACC, XROW), jnp.float32),
        mesh=mesh,
        scratch_types=scratch,
        compiler_params=pltpu.CompilerParams(
            needs_layout_passes=False, use_tc_tiling_on_sc=False),
    )
    return kern(xpad, eidx, dummy, zrows)


def _node_tc_kernel(x_ref, acc_ref, w1a_ref, w1b_ref, b1_ref, w2_ref, b2_ref,
                    o_ref):
    xb = x_ref[...]
    accs = acc_ref[0] + acc_ref[1]
    agg = accs[:, 0:5]
    num = accs[:, 5:7]
    den = accs[:, 7:8]
    h = jnp.maximum(
        jnp.dot(xb, w1a_ref[...], preferred_element_type=jnp.float32)
        + jnp.dot(agg, w1b_ref[...], preferred_element_type=jnp.float32)
        + b1_ref[...], 0.0)
    fdot = jnp.dot(h, w2_ref[...], preferred_element_type=jnp.float32) \
        + b2_ref[...]
    curv = num / (den + 1e-6)
    visc = DISS * jnp.tanh(jnp.abs(fdot)) * curv
    tot = jnp.clip(fdot + visc, -10.0, 10.0)
    o_ref[...] = jnp.clip(xb[:, 3:5] + DT * tot, -10.0, 10.0)


def _node_pass(x, partials, W1, b1, W2, b2):
    Bn = 1000
    grid = (N // Bn,)
    w1a = W1[:F]
    w1b = W1[F:]
    b1r = b1.reshape(1, H)
    b2r = b2.reshape(1, ND)
    return pl.pallas_call(
        _node_tc_kernel,
        grid=grid,
        in_specs=[
            pl.BlockSpec((Bn, F), lambda i: (i, 0)),
            pl.BlockSpec((NCORE, Bn, XROW), lambda i: (0, i, 0)),
            pl.BlockSpec((F, H), lambda i: (0, 0)),
            pl.BlockSpec((F, H), lambda i: (0, 0)),
            pl.BlockSpec((1, H), lambda i: (0, 0)),
            pl.BlockSpec((H, ND), lambda i: (0, 0)),
            pl.BlockSpec((1, ND), lambda i: (0, 0)),
        ],
        out_specs=pl.BlockSpec((Bn, ND), lambda i: (i, 0)),
        out_shape=jax.ShapeDtypeStruct((N, ND), jnp.float32),
    )(x, partials, w1a, w1b, b1r, W2, b2r)


def kernel(x, edge_index, W1, b1, W2, b2):
    E = edge_index.shape[1]
    chunk_span = NW * K * 2 * NBUF
    epad = chunk_span * ((E + chunk_span - 1) // chunk_span)
    ept = epad // NW
    nchunk = ept // K

    xpad = jnp.zeros((NACC, XROW), jnp.float32).at[:N, :F].set(x)
    dummy = jnp.full((2, K), N, jnp.int32)
    zrows = jnp.zeros((RPT, XROW), jnp.float32)

    partials = _edge_pass(xpad, edge_index, dummy, zrows, nchunk, ept, E)
    return _node_pass(x, partials, W1, b1, W2, b2)


# speedup vs baseline: 1.4058x; 1.4058x over previous
import functools

import jax
import jax.numpy as jnp
from jax import lax
from jax.experimental import pallas as pl
from jax.experimental.pallas import tpu as pltpu
from jax.experimental.pallas import tpu_sc as plsc

N = 50000
F = 5
ND = 2
H = 128
DISS = 0.01
DT = 1.0

NCORE = 2
NSUB = 16
NW = NCORE * NSUB
L = 16

K = 128
GPC = K // L
NBUF = 4
NACC = 50048
RPT = NACC // NSUB
XROW = 8


def _edge_sc_kernel(nchunk, ept, E, xpad, eidx, dummy, zrows, out, *bufs):
    IR = 2 * NBUF
    ibuf = bufs[0:IR]
    srows = bufs[IR:IR + NBUF]
    drows = bufs[IR + NBUF:IR + 2 * NBUF]
    semg = bufs[IR + 2 * NBUF:IR + 3 * NBUF]
    sems = bufs[IR + 3 * NBUF:IR + 4 * NBUF]
    semi = bufs[IR + 4 * NBUF:IR + 4 * NBUF + IR]
    acc = bufs[IR + 4 * NBUF + IR]

    c = lax.axis_index("c")
    s = lax.axis_index("s")
    wid = s * NCORE + c

    def issue_idx(ci, islot):
        base = wid * ept + ci * K
        @pl.when(base + K <= E)
        def _():
            pltpu.async_copy(eidx.at[:, pl.ds(base, K)], ibuf[islot],
                             semi[islot])
        @pl.when(base + K > E)
        def _():
            pltpu.async_copy(dummy, ibuf[islot], semi[islot])

    def wait_idx(islot):
        pltpu.make_async_copy(dummy, ibuf[islot], semi[islot]).wait()

    def issue_gathers(b, islot):
        pltpu.async_copy(xpad.at[ibuf[islot].at[0]], srows[b], semg[b])
        pltpu.async_copy(xpad.at[ibuf[islot].at[1]], drows[b], semg[b])

    def wait_gathers(b):
        pltpu.make_async_copy(xpad.at[ibuf[0].at[0]], srows[b], semg[b]).wait()
        pltpu.make_async_copy(xpad.at[ibuf[0].at[1]], drows[b], semg[b]).wait()

    def wait_scatter(b):
        pltpu.make_async_copy(srows[b], acc.at[ibuf[0].at[1]], sems[b]).wait()

    for j in range(NBUF):
        issue_idx(j, j)
    for j in range(2):
        wait_idx(j)
        issue_gathers(j, j)

    pltpu.sync_copy(zrows, acc.at[pl.ds(s * RPT, RPT)])
    plsc.subcore_barrier()

    def group_body(g, carry):
        for b in range(2 * NBUF):
            i = g * 2 * NBUF + b
            sb = b % NBUF
            wait_gathers(sb)
            for gg in range(GPC):
                rows = lax.iota(jnp.int32, L) + (gg * L)

                def col(ref, cc):
                    return plsc.load_gather(
                        ref, [rows, jnp.full((L,), cc, jnp.int32)])

                s0 = col(srows[sb], 0)
                s1 = col(srows[sb], 1)
                s3 = col(srows[sb], 3)
                s4 = col(srows[sb], 4)
                d0 = col(drows[sb], 0)
                d1 = col(drows[sb], 1)
                d3 = col(drows[sb], 3)
                d4 = col(drows[sb], 4)
                ex = s0 - d0
                ey = s1 - d1
                t = ex * ex + ey * ey
                bi = plsc.bitcast(t, jnp.int32)
                bi = jnp.int32(0x5F3759DF) - lax.shift_right_logical(bi, 1)
                y = plsc.bitcast(bi, jnp.float32)
                ht = t * 0.5
                y = y * (1.5 - ht * y * y)
                y = y * (1.5 - ht * y * y)
                y = y * (1.5 - ht * y * y)
                dist = t * y
                w = 1.0 / (dist + 1e-6)
                n0 = w * (s3 - d3)
                n1 = w * (s4 - d4)
                for cc, val in ((5, n0), (6, n1), (7, w)):
                    plsc.store_scatter(
                        srows[sb], [rows, jnp.full((L,), cc, jnp.int32)], val)
            pltpu.async_copy(srows[sb], acc.at[ibuf[b].at[1]], sems[sb],
                             add=True)
            if b >= 2:
                wait_scatter((b - 2) % NBUF)
            else:
                @pl.when(g > 0)
                def _():
                    wait_scatter((b - 2) % NBUF)
            @pl.when(i + 4 < nchunk)
            def _():
                issue_idx(i + 4, (b + 4) % IR)
            @pl.when(i + 2 < nchunk)
            def _():
                wait_idx((b + 2) % IR)
                issue_gathers((b + 2) % NBUF, (b + 2) % IR)
        return carry

    lax.fori_loop(0, nchunk // (2 * NBUF), group_body, 0)
    wait_scatter((nchunk - 2) % NBUF)
    wait_scatter((nchunk - 1) % NBUF)
    plsc.subcore_barrier()
    pltpu.sync_copy(acc.at[pl.ds(s * RPT, RPT)],
                    out.at[c].at[pl.ds(s * RPT, RPT)])


def _edge_pass(xpad, eidx, dummy, zrows, nchunk, ept, E):
    mesh = plsc.VectorSubcoreMesh(
        core_axis_name="c", subcore_axis_name="s",
        num_cores=NCORE, num_subcores=NSUB)
    IR = 2 * NBUF
    scratch = (
        [pltpu.VMEM((2, K), jnp.int32) for _ in range(IR)]
        + [pltpu.VMEM((K, XROW), jnp.float32) for _ in range(NBUF)]
        + [pltpu.VMEM((K, XROW), jnp.float32) for _ in range(NBUF)]
        + [pltpu.SemaphoreType.DMA for _ in range(NBUF)]
        + [pltpu.SemaphoreType.DMA for _ in range(NBUF)]
        + [pltpu.SemaphoreType.DMA for _ in range(IR)]
        + [pltpu.VMEM_SHARED((NACC, XROW), jnp.float32)]
    )
    kern = pl.kernel(
        functools.partial(_edge_sc_kernel, nchunk, ept, E),
        out_type=jax.ShapeDtypeStruct((NCORE, NACC, XROW), jnp.float32),
        mesh=mesh,
        scratch_types=scratch,
        compiler_params=pltpu.CompilerParams(
            needs_layout_passes=False, use_tc_tiling_on_sc=False),
    )
    return kern(xpad, eidx, dummy, zrows)


def _node_tc_kernel(x_ref, acc_ref, w1a_ref, w1b_ref, b1_ref, w2_ref, b2_ref,
                    o_ref):
    xb = x_ref[...]
    accs = acc_ref[0] + acc_ref[1]
    agg = accs[:, 0:5]
    num = accs[:, 5:7]
    den = accs[:, 7:8]
    h = jnp.maximum(
        jnp.dot(xb, w1a_ref[...], preferred_element_type=jnp.float32)
        + jnp.dot(agg, w1b_ref[...], preferred_element_type=jnp.float32)
        + b1_ref[...], 0.0)
    fdot = jnp.dot(h, w2_ref[...], preferred_element_type=jnp.float32) \
        + b2_ref[...]
    curv = num / (den + 1e-6)
    visc = DISS * jnp.tanh(jnp.abs(fdot)) * curv
    tot = jnp.clip(fdot + visc, -10.0, 10.0)
    o_ref[...] = jnp.clip(xb[:, 3:5] + DT * tot, -10.0, 10.0)


def _node_pass(x, partials, W1, b1, W2, b2):
    Bn = 5000
    grid = (N // Bn,)
    w1a = W1[:F]
    w1b = W1[F:]
    b1r = b1.reshape(1, H)
    b2r = b2.reshape(1, ND)
    return pl.pallas_call(
        _node_tc_kernel,
        grid=grid,
        in_specs=[
            pl.BlockSpec((Bn, F), lambda i: (i, 0)),
            pl.BlockSpec((NCORE, Bn, XROW), lambda i: (0, i, 0)),
            pl.BlockSpec((F, H), lambda i: (0, 0)),
            pl.BlockSpec((F, H), lambda i: (0, 0)),
            pl.BlockSpec((1, H), lambda i: (0, 0)),
            pl.BlockSpec((H, ND), lambda i: (0, 0)),
            pl.BlockSpec((1, ND), lambda i: (0, 0)),
        ],
        out_specs=pl.BlockSpec((Bn, ND), lambda i: (i, 0)),
        out_shape=jax.ShapeDtypeStruct((N, ND), jnp.float32),
    )(x, partials, w1a, w1b, b1r, W2, b2r)


def kernel(x, edge_index, W1, b1, W2, b2):
    E = edge_index.shape[1]
    chunk_span = NW * K * 2 * NBUF
    epad = chunk_span * ((E + chunk_span - 1) // chunk_span)
    ept = epad // NW
    nchunk = ept // K

    xpad = jnp.zeros((NACC, XROW), jnp.float32).at[:N, :F].set(x)
    dummy = jnp.full((2, K), N, jnp.int32)
    zrows = jnp.zeros((RPT, XROW), jnp.float32)

    partials = _edge_pass(xpad, edge_index, dummy, zrows, nchunk, ept, E)
    return _node_pass(x, partials, W1, b1, W2, b2)
